# Initial kernel scaffold; baseline (speedup 1.0000x reference)
import jax
import jax.numpy as jnp
from jax.experimental import pallas as pl


def kernel(x_q, scale_x, scale_out):
    raise NotImplementedError("write your pallas kernel here")


# speedup vs baseline: 1.5328x; 1.5328x over previous
import jax
import jax.numpy as jnp
from jax.experimental import pallas as pl
from jax.experimental.pallas import tpu as pltpu

QMIN, QMAX = -128, 127
BLOCK_ROWS = 256


def _softmax_block(x_ref, sx_ref, so_ref, o_ref):
    r = pl.program_id(1)
    x = x_ref[0].astype(jnp.float32)
    sx = sx_ref[0, 0]
    so = so_ref[0, 0]

    br, s = x.shape
    row = jax.lax.broadcasted_iota(jnp.int32, (br, s), 0) + r * br
    col = jax.lax.broadcasted_iota(jnp.int32, (br, s), 1)
    mask = col <= row

    logits = jnp.where(mask, x * sx, -1e30)
    m = jnp.max(logits, axis=-1, keepdims=True)
    e = jnp.exp(logits - m)
    denom = jnp.sum(e, axis=-1, keepdims=True)
    p = e / denom
    q = jnp.clip(jnp.round(p / so), QMIN, QMAX)
    o_ref[0] = q.astype(jnp.int8)


def kernel(x_q, scale_x, scale_out):
    h, s, _ = x_q.shape
    nb = s // BLOCK_ROWS
    sx4 = scale_x.reshape(h, nb, BLOCK_ROWS, 1)
    so4 = scale_out.reshape(h, nb, BLOCK_ROWS, 1)

    out_q = pl.pallas_call(
        _softmax_block,
        out_shape=jax.ShapeDtypeStruct((h, s, s), jnp.int8),
        grid=(h, nb),
        in_specs=[
            pl.BlockSpec((1, BLOCK_ROWS, s), lambda i, j: (i, j, 0)),
            pl.BlockSpec((1, 1, BLOCK_ROWS, 1), lambda i, j: (i, j, 0, 0)),
            pl.BlockSpec((1, 1, BLOCK_ROWS, 1), lambda i, j: (i, j, 0, 0)),
        ],
        out_specs=pl.BlockSpec((1, BLOCK_ROWS, s), lambda i, j: (i, j, 0)),
        compiler_params=pltpu.CompilerParams(
            dimension_semantics=("parallel", "arbitrary"),
        ),
        name="causal_softmax_quant",
    )(x_q, sx4, so4)
    return out_q, scale_out
